# baseline (speedup 1.0000x reference)
import functools

import numpy as np
import jax
import jax.numpy as jnp
from jax.experimental import pallas as pl
from jax.experimental.pallas import tpu as pltpu


_POOL_H = 112
_POOL_LANES = 128
_ROWS_PER_IMG = 3 * _POOL_H
_FEAT_DIM = 512


def _np_pool_matrix(out_size, in_size):
    i = np.arange(out_size)
    starts = (i * in_size) // out_size
    ends = ((i + 1) * in_size + out_size - 1) // out_size
    idx = np.arange(in_size)
    mask = (idx[None, :] >= starts[:, None]) & (idx[None, :] < ends[:, None])
    counts = (ends - starts).astype(np.float32)
    return mask.astype(np.float32) / counts[:, None]


@functools.lru_cache(maxsize=None)
def _combined_pool_mats(H, W):
    p112 = _np_pool_matrix(_POOL_H, 188)
    if H != 256:
        ph = _np_pool_matrix(256, H)
        pw = _np_pool_matrix(256, W)
    else:
        ph = np.eye(H, dtype=np.float32)
        pw = np.eye(W, dtype=np.float32)
    ah = p112 @ ph[35:223, :]
    aw = p112 @ pw[32:220, :]
    aw_pad = np.concatenate(
        [aw, np.zeros((_POOL_LANES - _POOL_H, W), np.float32)], axis=0)
    nz = np.nonzero(ah.any(axis=0))[0]
    r0 = (int(nz.min()) // 8) * 8
    r1 = min(H, -(-(int(nz.max()) + 1) // 8) * 8)
    return (jnp.asarray(ah[:, r0:r1], jnp.bfloat16),
            jnp.asarray(aw_pad.T, jnp.bfloat16),
            r0, r1)


@functools.lru_cache(maxsize=None)
def _mean_selector(n_imgs):
    sel = np.zeros((n_imgs, n_imgs * _ROWS_PER_IMG), np.float32)
    for i in range(n_imgs):
        sel[i, i * _ROWS_PER_IMG:(i + 1) * _ROWS_PER_IMG] = 1.0
    return jnp.asarray(sel)


def _fused_idloss(pred, gt, ah, awt, sel, w1, b1, alpha, w2, b2, pairs,
                  r0, r1):
    B, C, H, W = pred.shape
    inv_count = 1.0 / B
    n_imgs = 2 * pairs
    hc = r1 - r0

    def _body(xp_ref, xg_ref, ah_ref, awt_ref, sel_ref, w1_ref, b1_ref,
              a_ref, w2_ref, b2_ref, o_ref):
        @pl.when(pl.program_id(0) == 0)
        def _():
            o_ref[...] = jnp.zeros_like(o_ref)

        w1 = w1_ref[...].astype(jnp.bfloat16)
        w2 = w2_ref[...].astype(jnp.bfloat16)
        x3p = xp_ref[...].reshape(pairs * C, H, W)
        x3g = xg_ref[...].reshape(pairs * C, H, W)

        def _plane(x3, p):
            xs = x3[p, r0:r1, :].astype(jnp.bfloat16)
            t = jnp.dot(xs, awt_ref[...],
                        preferred_element_type=jnp.float32
                        ).astype(jnp.bfloat16)
            return jnp.dot(ah_ref[...], t,
                           preferred_element_type=jnp.float32
                           ).astype(jnp.bfloat16)

        pooled = jnp.concatenate(
            [_plane(x3, p) for x3 in (x3p, x3g) for p in range(pairs * C)],
            axis=0)
        h = jnp.dot(pooled, w1,
                    preferred_element_type=jnp.float32) + b1_ref[...]
        h = jnp.where(h > 0, h, a_ref[...] * h)
        m = (1.0 / _ROWS_PER_IMG) * jnp.dot(
            sel_ref[...], h, preferred_element_type=jnp.float32)
        f = jnp.dot(m.astype(jnp.bfloat16), w2,
                    preferred_element_type=jnp.float32) + b2_ref[...]
        ssq = jnp.sum(f * f, axis=-1, keepdims=True)
        fn = f * jax.lax.rsqrt(jnp.maximum(ssq, 1e-12))
        d = jnp.sum(fn[:pairs] * fn[pairs:], axis=-1, keepdims=True)
        o_ref[...] += jnp.sum((1.0 - d) * inv_count, keepdims=True)

    out = pl.pallas_call(
        _body,
        out_shape=jax.ShapeDtypeStruct((1, 1), jnp.float32),
        grid=(B // pairs,),
        in_specs=[
            pl.BlockSpec((pairs, C, H, W), lambda i: (i, 0, 0, 0)),
            pl.BlockSpec((pairs, C, H, W), lambda i: (i, 0, 0, 0)),
            pl.BlockSpec((_POOL_H, hc), lambda i: (0, 0)),
            pl.BlockSpec((W, _POOL_LANES), lambda i: (0, 0)),
            pl.BlockSpec((n_imgs, n_imgs * _ROWS_PER_IMG), lambda i: (0, 0)),
            pl.BlockSpec((_POOL_LANES, _FEAT_DIM), lambda i: (0, 0)),
            pl.BlockSpec((1, _FEAT_DIM), lambda i: (0, 0)),
            pl.BlockSpec((1, _FEAT_DIM), lambda i: (0, 0)),
            pl.BlockSpec((_FEAT_DIM, _FEAT_DIM), lambda i: (0, 0)),
            pl.BlockSpec((1, _FEAT_DIM), lambda i: (0, 0)),
        ],
        out_specs=pl.BlockSpec((1, 1), lambda i: (0, 0)),
        compiler_params=pltpu.CompilerParams(
            dimension_semantics=("arbitrary",)),
    )(pred, gt, ah, awt, sel, w1, b1, alpha, w2, b2)
    return out[0, 0]


@jax.jit
def kernel(pred, gt, w1, b1, alpha, w2, b2):
    B, C, H, W = pred.shape
    gt = jax.lax.stop_gradient(gt)
    pairs = 8 if B % 8 == 0 else (4 if B % 4 == 0 else 1)
    ah, awt, r0, r1 = _combined_pool_mats(H, W)
    sel = _mean_selector(2 * pairs)
    loss = _fused_idloss(pred, gt, ah, awt, sel, w1, b1, alpha, w2, b2, pairs,
                         r0, r1)
    return loss, jnp.float32(0.0)


# speedup vs baseline: 1.3671x; 1.3671x over previous
import functools

import numpy as np
import jax
import jax.numpy as jnp
from jax.experimental import pallas as pl
from jax.experimental.pallas import tpu as pltpu


_POOL_H = 112
_POOL_LANES = 128
_ROWS_PER_IMG = 3 * _POOL_H
_FEAT_DIM = 512


def _np_pool_matrix(out_size, in_size):
    i = np.arange(out_size)
    starts = (i * in_size) // out_size
    ends = ((i + 1) * in_size + out_size - 1) // out_size
    idx = np.arange(in_size)
    mask = (idx[None, :] >= starts[:, None]) & (idx[None, :] < ends[:, None])
    counts = (ends - starts).astype(np.float32)
    return mask.astype(np.float32) / counts[:, None]


@functools.lru_cache(maxsize=None)
def _combined_pool_mats(H, W):
    p112 = _np_pool_matrix(_POOL_H, 188)
    if H != 256:
        ph = _np_pool_matrix(256, H)
        pw = _np_pool_matrix(256, W)
    else:
        ph = np.eye(H, dtype=np.float32)
        pw = np.eye(W, dtype=np.float32)
    ah = p112 @ ph[35:223, :]
    aw = p112 @ pw[32:220, :]
    aw_pad = np.concatenate(
        [aw, np.zeros((_POOL_LANES - _POOL_H, W), np.float32)], axis=0)
    return (jnp.asarray(ah, jnp.bfloat16),
            jnp.asarray(aw_pad.T, jnp.bfloat16))


@functools.lru_cache(maxsize=None)
def _mean_selector(n_imgs):
    sel = np.zeros((n_imgs, n_imgs * _ROWS_PER_IMG), np.float32)
    for i in range(n_imgs):
        sel[i, i * _ROWS_PER_IMG:(i + 1) * _ROWS_PER_IMG] = 1.0
    return jnp.asarray(sel)


def _fused_idloss(pred, gt, ah, awt, sel, w1, b1, alpha, w2, b2, pairs):
    B, C, H, W = pred.shape
    threeH = C * H
    inv_count = 1.0 / B
    n_imgs = 2 * pairs
    imgs_per_chunk = sel.shape[0]

    def _body(xp_ref, xg_ref, ah_ref, awt_ref, sel_ref, w1_ref, b1_ref,
              a_ref, w2_ref, b2_ref, o_ref):
        @pl.when(pl.program_id(0) == 0)
        def _():
            o_ref[...] = jnp.zeros_like(o_ref)

        w1 = w1_ref[...].astype(jnp.bfloat16)
        w2 = w2_ref[...].astype(jnp.bfloat16)
        x2p = xp_ref[...].reshape(pairs * threeH, W)
        x2g = xg_ref[...].reshape(pairs * threeH, W)
        chunk_rows = imgs_per_chunk * threeH

        def _chunk(x2, k):
            xs = x2[k * chunk_rows:(k + 1) * chunk_rows, :].astype(
                jnp.bfloat16)
            t = jnp.dot(xs, awt_ref[...],
                        preferred_element_type=jnp.float32
                        ).astype(jnp.bfloat16)
            pooled = jnp.concatenate(
                [jnp.dot(ah_ref[...], t[c * H:(c + 1) * H, :],
                         preferred_element_type=jnp.float32
                         ).astype(jnp.bfloat16)
                 for c in range(3 * imgs_per_chunk)], axis=0)
            hk = jnp.dot(pooled, w1,
                         preferred_element_type=jnp.float32) + b1_ref[...]
            hk = jnp.where(hk > 0, hk, a_ref[...] * hk)
            return jnp.dot(sel_ref[...], hk,
                           preferred_element_type=jnp.float32)

        n_chunks = pairs // imgs_per_chunk
        m = (1.0 / _ROWS_PER_IMG) * jnp.concatenate(
            [_chunk(x2, k) for x2 in (x2p, x2g) for k in range(n_chunks)],
            axis=0)
        f = jnp.dot(m.astype(jnp.bfloat16), w2,
                    preferred_element_type=jnp.float32) + b2_ref[...]
        ssq = jnp.sum(f * f, axis=-1, keepdims=True)
        fn = f * jax.lax.rsqrt(jnp.maximum(ssq, 1e-12))
        d = jnp.sum(fn[:pairs] * fn[pairs:], axis=-1, keepdims=True)
        o_ref[...] += jnp.sum((1.0 - d) * inv_count, keepdims=True)

    out = pl.pallas_call(
        _body,
        out_shape=jax.ShapeDtypeStruct((1, 1), jnp.float32),
        grid=(B // pairs,),
        in_specs=[
            pl.BlockSpec((pairs, C, H, W), lambda i: (i, 0, 0, 0)),
            pl.BlockSpec((pairs, C, H, W), lambda i: (i, 0, 0, 0)),
            pl.BlockSpec((_POOL_H, H), lambda i: (0, 0)),
            pl.BlockSpec((W, _POOL_LANES), lambda i: (0, 0)),
            pl.BlockSpec(sel.shape, lambda i: (0, 0)),
            pl.BlockSpec((_POOL_LANES, _FEAT_DIM), lambda i: (0, 0)),
            pl.BlockSpec((1, _FEAT_DIM), lambda i: (0, 0)),
            pl.BlockSpec((1, _FEAT_DIM), lambda i: (0, 0)),
            pl.BlockSpec((_FEAT_DIM, _FEAT_DIM), lambda i: (0, 0)),
            pl.BlockSpec((1, _FEAT_DIM), lambda i: (0, 0)),
        ],
        out_specs=pl.BlockSpec((1, 1), lambda i: (0, 0)),
        compiler_params=pltpu.CompilerParams(
            dimension_semantics=("arbitrary",)),
    )(pred, gt, ah, awt, sel, w1, b1, alpha, w2, b2)
    return out[0, 0]


@jax.jit
def kernel(pred, gt, w1, b1, alpha, w2, b2):
    B, C, H, W = pred.shape
    gt = jax.lax.stop_gradient(gt)
    pairs = 8 if B % 8 == 0 else (4 if B % 4 == 0 else 1)
    ah, awt = _combined_pool_mats(H, W)
    sel = _mean_selector(2 if pairs % 2 == 0 else 1)
    loss = _fused_idloss(pred, gt, ah, awt, sel, w1, b1, alpha, w2, b2, pairs)
    return loss, jnp.float32(0.0)


# speedup vs baseline: 1.9686x; 1.4400x over previous
import functools

import numpy as np
import jax
import jax.numpy as jnp
from jax.experimental import pallas as pl
from jax.experimental.pallas import tpu as pltpu


_POOL_H = 112
_POOL_LANES = 128
_ROWS_PER_IMG = 3 * _POOL_H
_FEAT_DIM = 512


def _np_pool_matrix(out_size, in_size):
    i = np.arange(out_size)
    starts = (i * in_size) // out_size
    ends = ((i + 1) * in_size + out_size - 1) // out_size
    idx = np.arange(in_size)
    mask = (idx[None, :] >= starts[:, None]) & (idx[None, :] < ends[:, None])
    counts = (ends - starts).astype(np.float32)
    return mask.astype(np.float32) / counts[:, None]


@functools.lru_cache(maxsize=None)
def _combined_pool_mats(H, W):
    p112 = _np_pool_matrix(_POOL_H, 188)
    if H != 256:
        ph = _np_pool_matrix(256, H)
        pw = _np_pool_matrix(256, W)
    else:
        ph = np.eye(H, dtype=np.float32)
        pw = np.eye(W, dtype=np.float32)
    ah = p112 @ ph[35:223, :]
    aw = p112 @ pw[32:220, :]
    aw_pad = np.concatenate(
        [aw, np.zeros((_POOL_LANES - _POOL_H, W), np.float32)], axis=0)
    return (jnp.asarray(ah, jnp.bfloat16),
            jnp.asarray(aw_pad.T, jnp.bfloat16))


@functools.lru_cache(maxsize=None)
def _mean_selector(n_imgs):
    sel = np.zeros((n_imgs, n_imgs * _ROWS_PER_IMG), np.float32)
    for i in range(n_imgs):
        sel[i, i * _ROWS_PER_IMG:(i + 1) * _ROWS_PER_IMG] = 1.0
    return jnp.asarray(sel)


def _fused_idloss(pred, gt, ah, awt, sel, w1, b1, alpha, w2, b2, pairs):
    B, C, H, W = pred.shape
    threeH = C * H
    inv_count = 1.0 / B
    n_imgs = 2 * pairs
    planes = 3 * n_imgs

    def _body(xp_ref, xg_ref, ah_ref, awt_ref, sel_ref, w1_ref, b1_ref,
              a_ref, w2_ref, b2_ref, o_ref):
        @pl.when(pl.program_id(0) == 0)
        def _():
            o_ref[...] = jnp.zeros_like(o_ref)

        xp = xp_ref[...].reshape(pairs * threeH, W).astype(jnp.bfloat16)
        xg = xg_ref[...].reshape(pairs * threeH, W).astype(jnp.bfloat16)
        w1 = w1_ref[...].astype(jnp.bfloat16)
        w2 = w2_ref[...].astype(jnp.bfloat16)
        tp = jnp.dot(xp, awt_ref[...],
                     preferred_element_type=jnp.float32).astype(jnp.bfloat16)
        tg = jnp.dot(xg, awt_ref[...],
                     preferred_element_type=jnp.float32).astype(jnp.bfloat16)
        pooled = jnp.concatenate(
            [jnp.dot(ah_ref[...], t[c * H:(c + 1) * H, :],
                     preferred_element_type=jnp.float32).astype(jnp.bfloat16)
             for t in (tp, tg) for c in range(planes // 2)],
            axis=0)
        h = jnp.dot(pooled, w1,
                    preferred_element_type=jnp.float32) + b1_ref[...]
        h = jnp.where(h > 0, h, a_ref[...] * h)
        m = (1.0 / _ROWS_PER_IMG) * jnp.dot(
            sel_ref[...], h, preferred_element_type=jnp.float32)
        f = jnp.dot(m.astype(jnp.bfloat16), w2,
                    preferred_element_type=jnp.float32) + b2_ref[...]
        ssq = jnp.sum(f * f, axis=-1, keepdims=True)
        fn = f * jax.lax.rsqrt(jnp.maximum(ssq, 1e-12))
        d = jnp.sum(fn[:pairs] * fn[pairs:], axis=-1, keepdims=True)
        o_ref[...] += jnp.sum((1.0 - d) * inv_count, keepdims=True)

    out = pl.pallas_call(
        _body,
        out_shape=jax.ShapeDtypeStruct((1, 1), jnp.float32),
        grid=(B // pairs,),
        in_specs=[
            pl.BlockSpec((pairs, C, H, W), lambda i: (i, 0, 0, 0)),
            pl.BlockSpec((pairs, C, H, W), lambda i: (i, 0, 0, 0)),
            pl.BlockSpec((_POOL_H, H), lambda i: (0, 0)),
            pl.BlockSpec((W, _POOL_LANES), lambda i: (0, 0)),
            pl.BlockSpec((n_imgs, n_imgs * _ROWS_PER_IMG), lambda i: (0, 0)),
            pl.BlockSpec((_POOL_LANES, _FEAT_DIM), lambda i: (0, 0)),
            pl.BlockSpec((1, _FEAT_DIM), lambda i: (0, 0)),
            pl.BlockSpec((1, _FEAT_DIM), lambda i: (0, 0)),
            pl.BlockSpec((_FEAT_DIM, _FEAT_DIM), lambda i: (0, 0)),
            pl.BlockSpec((1, _FEAT_DIM), lambda i: (0, 0)),
        ],
        out_specs=pl.BlockSpec((1, 1), lambda i: (0, 0)),
        compiler_params=pltpu.CompilerParams(
            dimension_semantics=("arbitrary",),
            vmem_limit_bytes=60000 * 1024),
    )(pred, gt, ah, awt, sel, w1, b1, alpha, w2, b2)
    return out[0, 0]


@jax.jit
def kernel(pred, gt, w1, b1, alpha, w2, b2):
    B, C, H, W = pred.shape
    gt = jax.lax.stop_gradient(gt)
    pairs = 8 if B % 8 == 0 else (4 if B % 4 == 0 else 1)
    ah, awt = _combined_pool_mats(H, W)
    sel = _mean_selector(2 * pairs)
    loss = _fused_idloss(pred, gt, ah, awt, sel, w1, b1, alpha, w2, b2, pairs)
    return loss, jnp.float32(0.0)


# speedup vs baseline: 2.0313x; 1.0318x over previous
import functools

import numpy as np
import jax
import jax.numpy as jnp
from jax.experimental import pallas as pl
from jax.experimental.pallas import tpu as pltpu


_POOL_H = 112
_POOL_LANES = 128
_ROWS_PER_IMG = 3 * _POOL_H
_FEAT_DIM = 512


def _np_pool_matrix(out_size, in_size):
    i = np.arange(out_size)
    starts = (i * in_size) // out_size
    ends = ((i + 1) * in_size + out_size - 1) // out_size
    idx = np.arange(in_size)
    mask = (idx[None, :] >= starts[:, None]) & (idx[None, :] < ends[:, None])
    counts = (ends - starts).astype(np.float32)
    return mask.astype(np.float32) / counts[:, None]


@functools.lru_cache(maxsize=None)
def _combined_pool_mats(H, W):
    p112 = _np_pool_matrix(_POOL_H, 188)
    if H != 256:
        ph = _np_pool_matrix(256, H)
        pw = _np_pool_matrix(256, W)
    else:
        ph = np.eye(H, dtype=np.float32)
        pw = np.eye(W, dtype=np.float32)
    ah = p112 @ ph[35:223, :]
    aw = p112 @ pw[32:220, :]
    aw_pad = np.concatenate(
        [aw, np.zeros((_POOL_LANES - _POOL_H, W), np.float32)], axis=0)
    return (jnp.asarray(ah, jnp.bfloat16),
            jnp.asarray(aw_pad.T, jnp.bfloat16))


@functools.lru_cache(maxsize=None)
def _mean_selector(n_imgs):
    sel = np.zeros((n_imgs, n_imgs * _ROWS_PER_IMG), np.float32)
    for i in range(n_imgs):
        sel[i, i * _ROWS_PER_IMG:(i + 1) * _ROWS_PER_IMG] = 1.0
    return jnp.asarray(sel)


def _fused_idloss(pred, gt, ah, awt, sel, w1, b1, alpha, w2, b2, pairs):
    B, C, H, W = pred.shape
    threeH = C * H
    inv_count = 1.0 / B
    n_imgs = 2 * pairs
    planes = 3 * n_imgs

    def _body(xp_ref, xg_ref, ah_ref, awt_ref, sel_ref, w1_ref, b1_ref,
              a_ref, w2_ref, b2_ref, o_ref):
        @pl.when(pl.program_id(0) == 0)
        def _():
            o_ref[...] = jnp.zeros_like(o_ref)

        xp = xp_ref[...].reshape(pairs * threeH, W).astype(jnp.bfloat16)
        xg = xg_ref[...].reshape(pairs * threeH, W).astype(jnp.bfloat16)
        row = jax.lax.broadcasted_iota(jnp.int32, (_POOL_LANES, _FEAT_DIM), 0)
        w1 = jnp.where(row == _POOL_H, b1_ref[...], w1_ref[...]
                       ).astype(jnp.bfloat16)
        w2 = w2_ref[...].astype(jnp.bfloat16)
        tp = jnp.dot(xp, awt_ref[...],
                     preferred_element_type=jnp.float32).astype(jnp.bfloat16)
        tg = jnp.dot(xg, awt_ref[...],
                     preferred_element_type=jnp.float32).astype(jnp.bfloat16)
        pooled = jnp.concatenate(
            [jnp.dot(ah_ref[...], t[c * H:(c + 1) * H, :],
                     preferred_element_type=jnp.float32).astype(jnp.bfloat16)
             for t in (tp, tg) for c in range(planes // 2)],
            axis=0)
        lane = jax.lax.broadcasted_iota(
            jnp.int32, (n_imgs * _ROWS_PER_IMG, _POOL_LANES), 1)
        pooled = jnp.where(lane == _POOL_H, jnp.bfloat16(1.0), pooled)
        h = jnp.dot(pooled, w1,
                    preferred_element_type=jnp.float32)
        h = jnp.where(h > 0, h, a_ref[...] * h)
        m = (1.0 / _ROWS_PER_IMG) * jnp.dot(
            sel_ref[...], h, preferred_element_type=jnp.float32)
        f = jnp.dot(m.astype(jnp.bfloat16), w2,
                    preferred_element_type=jnp.float32) + b2_ref[...]
        ssq = jnp.sum(f * f, axis=-1, keepdims=True)
        fn = f * jax.lax.rsqrt(jnp.maximum(ssq, 1e-12))
        d = jnp.sum(fn[:pairs] * fn[pairs:], axis=-1, keepdims=True)
        o_ref[...] += jnp.sum((1.0 - d) * inv_count, keepdims=True)

    out = pl.pallas_call(
        _body,
        out_shape=jax.ShapeDtypeStruct((1, 1), jnp.float32),
        grid=(B // pairs,),
        in_specs=[
            pl.BlockSpec((pairs, C, H, W), lambda i: (i, 0, 0, 0)),
            pl.BlockSpec((pairs, C, H, W), lambda i: (i, 0, 0, 0)),
            pl.BlockSpec((_POOL_H, H), lambda i: (0, 0)),
            pl.BlockSpec((W, _POOL_LANES), lambda i: (0, 0)),
            pl.BlockSpec((n_imgs, n_imgs * _ROWS_PER_IMG), lambda i: (0, 0)),
            pl.BlockSpec((_POOL_LANES, _FEAT_DIM), lambda i: (0, 0)),
            pl.BlockSpec((1, _FEAT_DIM), lambda i: (0, 0)),
            pl.BlockSpec((1, _FEAT_DIM), lambda i: (0, 0)),
            pl.BlockSpec((_FEAT_DIM, _FEAT_DIM), lambda i: (0, 0)),
            pl.BlockSpec((1, _FEAT_DIM), lambda i: (0, 0)),
        ],
        out_specs=pl.BlockSpec((1, 1), lambda i: (0, 0)),
        compiler_params=pltpu.CompilerParams(
            dimension_semantics=("arbitrary",),
            vmem_limit_bytes=60000 * 1024),
    )(pred, gt, ah, awt, sel, w1, b1, alpha, w2, b2)
    return out[0, 0]


@jax.jit
def kernel(pred, gt, w1, b1, alpha, w2, b2):
    B, C, H, W = pred.shape
    gt = jax.lax.stop_gradient(gt)
    pairs = 8 if B % 8 == 0 else (4 if B % 4 == 0 else 1)
    ah, awt = _combined_pool_mats(H, W)
    sel = _mean_selector(2 * pairs)
    loss = _fused_idloss(pred, gt, ah, awt, sel, w1, b1, alpha, w2, b2, pairs)
    return loss, jnp.float32(0.0)
